# baseline (speedup 1.0000x reference)
import functools

import jax
import jax.numpy as jnp
from jax import lax
from jax.experimental import pallas as pl
from jax.experimental.pallas import tpu as pltpu
from jax.experimental.pallas import tpu_sc as plsc

_NUM_GRAPHS = 64
_NW = 32
_CHUNK = 2000
_ACC_W = 32


def _sc_edge_kernel(src_hbm, attrf_hbm, batch_hbm, out_hbm,
                    batch_v, src_v, attr_v, acc, part):
    ew = src_hbm.shape[0] // _NW
    nchunks = ew // _CHUNK
    cid = lax.axis_index("c")
    sid = lax.axis_index("s")
    wid = sid * 2 + cid
    base = wid * ew

    pltpu.sync_copy(batch_hbm, batch_v)

    zeros16 = jnp.zeros((16,), jnp.float32)

    def _zero_row(r, _):
        acc[pl.ds(r * 16, 16)] = zeros16
        return 0
    lax.fori_loop(0, 16 * _NUM_GRAPHS * _ACC_W // 16, _zero_row, 0)

    lane = lax.broadcasted_iota(jnp.int32, (16,), 0)
    lane16 = lane * 16
    lane_acc = lane * (_NUM_GRAPHS * _ACC_W)
    ones16 = jnp.ones((16,), jnp.float32)

    def _group(j, _):
        src16 = src_v[pl.ds(j * 16, 16)]
        seg16 = plsc.load_gather(batch_v, [src16])
        abase = lane16 + j * 256
        sbase = lane_acc + seg16 * _ACC_W
        for c in range(16):
            vals = plsc.load_gather(attr_v, [abase + c])
            plsc.addupdate_scatter(acc, [sbase + c], vals)
        plsc.addupdate_scatter(acc, [sbase + 16], ones16)
        return 0

    for k in range(nchunks):
        off = base + k * _CHUNK
        pltpu.sync_copy(src_hbm.at[pl.ds(off, _CHUNK)], src_v)
        pltpu.sync_copy(attrf_hbm.at[pl.ds(off * 16, _CHUNK * 16)], attr_v)
        lax.fori_loop(0, _CHUNK // 16, _group, 0)

    def _reduce_row(g, _):
        lo = acc[pl.ds(g * _ACC_W, 16)]
        hi = acc[pl.ds(g * _ACC_W + 16, 16)]
        for l in range(1, 16):
            o = l * (_NUM_GRAPHS * _ACC_W) + g * _ACC_W
            lo = lo + acc[pl.ds(o, 16)]
            hi = hi + acc[pl.ds(o + 16, 16)]
        part[pl.ds(g * _ACC_W, 16)] = lo
        part[pl.ds(g * _ACC_W + 16, 16)] = hi
        return 0
    lax.fori_loop(0, _NUM_GRAPHS, _reduce_row, 0)

    pltpu.sync_copy(part, out_hbm.at[wid])


def _sc_edge_sums(src, edge_attr, batch):
    E = src.shape[0]
    N = batch.shape[0]
    mesh = plsc.VectorSubcoreMesh(core_axis_name="c", subcore_axis_name="s")
    kern = functools.partial(
        pl.kernel, mesh=mesh,
        out_type=jax.ShapeDtypeStruct((_NW, _NUM_GRAPHS * _ACC_W),
                                      jnp.float32),
        scratch_types=[
            pltpu.VMEM((N,), jnp.int32),
            pltpu.VMEM((_CHUNK,), jnp.int32),
            pltpu.VMEM((_CHUNK * 16,), jnp.float32),
            pltpu.VMEM((16 * _NUM_GRAPHS * _ACC_W,), jnp.float32),
            pltpu.VMEM((_NUM_GRAPHS * _ACC_W,), jnp.float32),
        ],
        compiler_params=pltpu.CompilerParams(needs_layout_passes=False),
    )(_sc_edge_kernel)
    return kern(src, edge_attr.reshape(E * 16), batch)


def _tc_body(psc_ref, batch_ref, x_ref, u_ref, w1u_ref, w1n_ref,
             w1e_ref, b1_ref, w2_ref, b2_ref, wa_ref, ba_ref, out_ref):
    tot = psc_ref[0]
    for i in range(1, _NW):
        tot = tot + psc_ref[i]
    esum = tot[:, :16]
    ecnt = tot[:, 16:17]
    emean = esum / jnp.maximum(ecnt, 1.0)

    b = batch_ref[...]
    gcol = lax.broadcasted_iota(jnp.int32, (_NUM_GRAPHS, 1), 0)
    onehot = (b == gcol).astype(jnp.float32)
    nsum = jnp.dot(onehot, x_ref[...], preferred_element_type=jnp.float32)
    ncnt = jnp.sum(onehot, axis=1, keepdims=True)
    nmean = nsum / jnp.maximum(ncnt, 1.0)

    h = (jnp.dot(u_ref[...], w1u_ref[...], preferred_element_type=jnp.float32)
         + jnp.dot(nmean, w1n_ref[...], preferred_element_type=jnp.float32)
         + jnp.dot(emean, w1e_ref[...], preferred_element_type=jnp.float32)
         + b1_ref[...])
    h = jnp.maximum(h, 0.0)
    g = jnp.dot(h, w2_ref[...], preferred_element_type=jnp.float32) \
        + b2_ref[...]
    a = jax.nn.sigmoid(jnp.dot(g, wa_ref[...],
                               preferred_element_type=jnp.float32)
                       + ba_ref[...])
    out_ref[...] = g * a


def kernel(x, edge_index, edge_attr, u, batch, W1, b1, W2, b2, Wa, ba):
    N, node_dim = x.shape
    E, edge_dim = edge_attr.shape
    global_dim = u.shape[1]
    hidden_dim = W1.shape[1]

    src = edge_index[0].astype(jnp.int32)
    batch_i32 = batch.astype(jnp.int32)

    psc = _sc_edge_sums(src, edge_attr, batch_i32)
    psc = psc.reshape(_NW, _NUM_GRAPHS, _ACC_W)

    batch2d = batch_i32.reshape(1, N)
    w1u = W1[:global_dim]
    w1n = W1[global_dim:global_dim + node_dim]
    w1e = W1[global_dim + node_dim:]
    b1r = b1.reshape(1, hidden_dim)
    b2r = b2.reshape(1, global_dim)
    bar = ba.reshape(1, 1)

    return pl.pallas_call(
        _tc_body,
        out_shape=jax.ShapeDtypeStruct((_NUM_GRAPHS, global_dim),
                                       jnp.float32),
    )(psc, batch2d, x, u, w1u, w1n, w1e, b1r, W2, b2r, Wa, bar)


# speedup vs baseline: 1.8282x; 1.8282x over previous
import functools

import jax
import jax.numpy as jnp
from jax import lax
from jax.experimental import pallas as pl
from jax.experimental.pallas import tpu as pltpu
from jax.experimental.pallas import tpu_sc as plsc

_NUM_GRAPHS = 64
_NW = 32
_CHUNK = 2000
_WIN = 80
_ACC_W = 32


def _sc_edge_kernel(src_hbm, attr_hbm, batch_hbm, sums_hbm, cnts_hbm,
                    batch_v, src_v, attr_v, seg2d, cntacc, part, zbuf,
                    acc_sh):
    ew = src_hbm.shape[0] // _NW
    nchunks = ew // _CHUNK
    nwin = _CHUNK // _WIN
    cid = lax.axis_index("c")
    sid = lax.axis_index("s")
    wid = sid * 2 + cid
    base = wid * ew

    zeros16 = jnp.zeros((16,), jnp.float32)
    lane = lax.broadcasted_iota(jnp.int32, (16,), 0)
    ones16 = jnp.ones((16,), jnp.float32)
    lane64 = lane * _NUM_GRAPHS

    pltpu.sync_copy(batch_hbm, batch_v)
    for r in range(16 * _NUM_GRAPHS // 16):
        cntacc[pl.ds(r * 16, 16)] = zeros16
    for r in range(_NUM_GRAPHS * _ACC_W // 16):
        part[pl.ds(r * 16, 16)] = zeros16

    @pl.when(sid == 0)
    def _init_shared():
        for r in range(_NUM_GRAPHS):
            zbuf[r, pl.ds(0, 16)] = zeros16
        pltpu.sync_copy(zbuf, acc_sh)

    plsc.subcore_barrier()

    def _window(w, _):
        for g in range(_WIN // 16):
            src16 = src_v[pl.ds(w * _WIN + g * 16, 16)]
            seg16 = plsc.load_gather(batch_v, [src16])
            seg2d[w, pl.ds(g * 16, 16)] = seg16
            plsc.addupdate_scatter(cntacc, [lane64 + seg16], ones16)
        pltpu.sync_copy(attr_v.at[pl.ds(w * _WIN, _WIN)],
                        acc_sh.at[seg2d.at[w]], add=True)
        return 0

    for k in range(nchunks):
        off = base + k * _CHUNK
        pltpu.sync_copy(src_hbm.at[pl.ds(off, _CHUNK)], src_v)
        pltpu.sync_copy(attr_hbm.at[pl.ds(off, _CHUNK)], attr_v)
        lax.fori_loop(0, nwin, _window, 0)

    for gb in range(_NUM_GRAPHS // 16):
        tot = cntacc[pl.ds(gb * 16, 16)]
        for l in range(1, 16):
            tot = tot + cntacc[pl.ds(l * _NUM_GRAPHS + gb * 16, 16)]
        pidx = (gb * 16 + lane) * _ACC_W + 16
        plsc.store_scatter(part, [pidx], tot)
    pltpu.sync_copy(part, cnts_hbm.at[wid])

    plsc.subcore_barrier()

    @pl.when(sid == 0)
    def _flush_shared():
        pltpu.sync_copy(acc_sh, sums_hbm.at[cid])


def _sc_edge_sums(src, edge_attr, batch):
    E = src.shape[0]
    N = batch.shape[0]
    mesh = plsc.VectorSubcoreMesh(core_axis_name="c", subcore_axis_name="s")
    kern = functools.partial(
        pl.kernel, mesh=mesh,
        out_type=(
            jax.ShapeDtypeStruct((2, _NUM_GRAPHS, 16), jnp.float32),
            jax.ShapeDtypeStruct((_NW, _NUM_GRAPHS * _ACC_W), jnp.float32),
        ),
        scratch_types=[
            pltpu.VMEM((N,), jnp.int32),
            pltpu.VMEM((_CHUNK,), jnp.int32),
            pltpu.VMEM((_CHUNK, 16), jnp.float32),
            pltpu.VMEM((_CHUNK // _WIN, _WIN), jnp.int32),
            pltpu.VMEM((16 * _NUM_GRAPHS,), jnp.float32),
            pltpu.VMEM((_NUM_GRAPHS * _ACC_W,), jnp.float32),
            pltpu.VMEM((_NUM_GRAPHS, 16), jnp.float32),
            pltpu.VMEM_SHARED((_NUM_GRAPHS, 16), jnp.float32),
        ],
        compiler_params=pltpu.CompilerParams(
            needs_layout_passes=False,
            use_tc_tiling_on_sc=False,
        ),
    )(_sc_edge_kernel)
    return kern(src, edge_attr, batch)


def _tc_body(sums_ref, psc_ref, batch_ref, x_ref, u_ref, w1u_ref, w1n_ref,
             w1e_ref, b1_ref, w2_ref, b2_ref, wa_ref, ba_ref, out_ref):
    esum = sums_ref[0] + sums_ref[1]
    tot = psc_ref[0]
    for i in range(1, _NW):
        tot = tot + psc_ref[i]
    ecnt = tot[:, 16:17]
    emean = esum / jnp.maximum(ecnt, 1.0)

    b = batch_ref[...]
    gcol = lax.broadcasted_iota(jnp.int32, (_NUM_GRAPHS, 1), 0)
    onehot = (b == gcol).astype(jnp.float32)
    nsum = jnp.dot(onehot, x_ref[...], preferred_element_type=jnp.float32)
    ncnt = jnp.sum(onehot, axis=1, keepdims=True)
    nmean = nsum / jnp.maximum(ncnt, 1.0)

    h = (jnp.dot(u_ref[...], w1u_ref[...], preferred_element_type=jnp.float32)
         + jnp.dot(nmean, w1n_ref[...], preferred_element_type=jnp.float32)
         + jnp.dot(emean, w1e_ref[...], preferred_element_type=jnp.float32)
         + b1_ref[...])
    h = jnp.maximum(h, 0.0)
    g = jnp.dot(h, w2_ref[...], preferred_element_type=jnp.float32) \
        + b2_ref[...]
    a = jax.nn.sigmoid(jnp.dot(g, wa_ref[...],
                               preferred_element_type=jnp.float32)
                       + ba_ref[...])
    out_ref[...] = g * a


def kernel(x, edge_index, edge_attr, u, batch, W1, b1, W2, b2, Wa, ba):
    N, node_dim = x.shape
    E, edge_dim = edge_attr.shape
    global_dim = u.shape[1]
    hidden_dim = W1.shape[1]

    src = edge_index[0].astype(jnp.int32)
    batch_i32 = batch.astype(jnp.int32)

    sums, psc = _sc_edge_sums(src, edge_attr, batch_i32)
    psc = psc.reshape(_NW, _NUM_GRAPHS, _ACC_W)

    batch2d = batch_i32.reshape(1, N)
    w1u = W1[:global_dim]
    w1n = W1[global_dim:global_dim + node_dim]
    w1e = W1[global_dim + node_dim:]
    b1r = b1.reshape(1, hidden_dim)
    b2r = b2.reshape(1, global_dim)
    bar = ba.reshape(1, 1)

    return pl.pallas_call(
        _tc_body,
        out_shape=jax.ShapeDtypeStruct((_NUM_GRAPHS, global_dim),
                                       jnp.float32),
    )(sums, psc, batch2d, x, u, w1u, w1n, w1e, b1r, W2, b2r, Wa, bar)
